# baseline (speedup 1.0000x reference)
import functools

import jax
import jax.numpy as jnp
from jax import lax
from jax.experimental import pallas as pl
from jax.experimental.pallas import tpu as pltpu
from jax.experimental.pallas import tpu_sc as plsc

CLASSES = 100000
SMOOTHING = 0.1
CONFIDENCE = 1.0 - SMOOTHING
EPS = SMOOTHING / (CLASSES - 1)
N_ROWS = 1024


_NC = 2
_NS = 16
_NW = _NC * _NS
_RPW = N_ROWS // _NW
_LANES = 16
_CHUNK = 128


def _sc_gather_kernel(pred_hbm, tgt_hbm, out_hbm, tgt_v, rows_v, sem):
    wid = lax.axis_index("s") * _NC + lax.axis_index("c")
    base = wid * _RPW
    pltpu.sync_copy(tgt_hbm.at[pl.ds(base, _RPW)], tgt_v)
    copies = []
    for c in range(_RPW // _LANES):
        tv = tgt_v[pl.ds(c * _LANES, _LANES)]
        sv = jnp.minimum(tv & -_CHUNK, CLASSES - _CHUNK)
        for j in range(_LANES):
            r = c * _LANES + j
            start = pl.multiple_of(sv[j], _CHUNK)
            copies.append(pltpu.async_copy(
                pred_hbm.at[base + r, pl.ds(start, _CHUNK)],
                rows_v.at[r], sem))
    for cp in copies:
        cp.wait()
    pltpu.sync_copy(rows_v, out_hbm.at[pl.ds(base, _RPW)])


_sc_gather = functools.partial(
    pl.kernel,
    mesh=plsc.VectorSubcoreMesh(core_axis_name="c", subcore_axis_name="s"),
    out_type=jax.ShapeDtypeStruct((N_ROWS, _CHUNK), jnp.float32),
    scratch_types=[
        pltpu.VMEM((_RPW,), jnp.int32),
        pltpu.VMEM((_RPW, _CHUNK), jnp.float32),
        pltpu.SemaphoreType.DMA,
    ],
)(_sc_gather_kernel)


C_BLK = 3072
N_BLK = (CLASSES + C_BLK - 1) // C_BLK


def _loss_kernel(tgt_ref, chunk_ref, pred_ref, out_ref, acc_ref):
    i = pl.program_id(0)
    x = pred_ref[...]

    def accumulate(partial):
        @pl.when(i == 0)
        def _init():
            acc_ref[...] = partial

        @pl.when(i > 0)
        def _accum():
            acc_ref[...] += partial

    def partials(xe, xs):
        sumexp = jnp.sum(jnp.exp(xe), axis=1, keepdims=True)
        sumpred = jnp.sum(xs, axis=1, keepdims=True)
        return jnp.concatenate([sumexp, sumpred], axis=1)

    @pl.when(i < N_BLK - 1)
    def _full_block():
        accumulate(partials(x, x))

    @pl.when(i == N_BLK - 1)
    def _tail_block():
        cols = jax.lax.broadcasted_iota(jnp.int32, x.shape, 1) + i * C_BLK
        valid = cols < CLASSES
        accumulate(partials(jnp.where(valid, x, -jnp.inf),
                            jnp.where(valid, x, 0.0)))

    @pl.when(i == N_BLK - 1)
    def _finalize():
        acc = acc_ref[...]
        t = tgt_ref[...]
        lane = t - jnp.minimum(t & -_CHUNK, CLASSES - _CHUNK)
        lanes = jax.lax.broadcasted_iota(jnp.int32, (N_ROWS, _CHUNK), 1)
        tgtval = jnp.sum(jnp.where(lanes == lane, chunk_ref[...], 0.0),
                         axis=1, keepdims=True)
        rows = (jnp.log(acc[:, 0:1]) - EPS * acc[:, 1:2]
                - (CONFIDENCE - EPS) * tgtval)
        out_ref[0, 0] = jnp.sum(rows) / N_ROWS


@jax.jit
def _run(pred, target):
    tgt32 = target.astype(jnp.int32)
    chunks = _sc_gather(pred, tgt32)
    out = pl.pallas_call(
        _loss_kernel,
        grid=(N_BLK,),
        in_specs=[
            pl.BlockSpec((N_ROWS, 1), lambda i: (0, 0)),
            pl.BlockSpec((N_ROWS, _CHUNK), lambda i: (0, 0)),
            pl.BlockSpec((N_ROWS, C_BLK), lambda i: (0, i)),
        ],
        out_specs=pl.BlockSpec((1, 1), lambda i: (0, 0),
                               memory_space=pltpu.SMEM),
        out_shape=jax.ShapeDtypeStruct((1, 1), jnp.float32),
        scratch_shapes=[pltpu.VMEM((N_ROWS, 2), jnp.float32)],
        compiler_params=pltpu.CompilerParams(
            dimension_semantics=("arbitrary",),
        ),
    )(tgt32.reshape(N_ROWS, 1), chunks, pred)
    return out[0, 0]


def kernel(pred, target):
    return _run(pred, target)


# speedup vs baseline: 1.0115x; 1.0115x over previous
import functools

import jax
import jax.numpy as jnp
from jax import lax
from jax.experimental import pallas as pl
from jax.experimental.pallas import tpu as pltpu
from jax.experimental.pallas import tpu_sc as plsc

CLASSES = 100000
SMOOTHING = 0.1
CONFIDENCE = 1.0 - SMOOTHING
EPS = SMOOTHING / (CLASSES - 1)
N_ROWS = 1024


_NC = 2
_NS = 16
_NW = _NC * _NS
_RPW = N_ROWS // _NW
_LANES = 16
_CHUNK = 128


def _sc_gather_kernel(pred_hbm, tgt_hbm, out_hbm, tgt_v, rows_v, sem):
    wid = lax.axis_index("s") * _NC + lax.axis_index("c")
    base = wid * _RPW
    pltpu.sync_copy(tgt_hbm.at[pl.ds(base, _RPW)], tgt_v)
    copies = []
    for c in range(_RPW // _LANES):
        tv = tgt_v[pl.ds(c * _LANES, _LANES)]
        sv = jnp.minimum(tv & -_CHUNK, CLASSES - _CHUNK)
        for j in range(_LANES):
            r = c * _LANES + j
            start = pl.multiple_of(sv[j], _CHUNK)
            copies.append(pltpu.async_copy(
                pred_hbm.at[base + r, pl.ds(start, _CHUNK)],
                rows_v.at[r], sem))
    for cp in copies:
        cp.wait()
    pltpu.sync_copy(rows_v, out_hbm.at[pl.ds(base, _RPW)])


_sc_gather = functools.partial(
    pl.kernel,
    mesh=plsc.VectorSubcoreMesh(core_axis_name="c", subcore_axis_name="s"),
    out_type=jax.ShapeDtypeStruct((N_ROWS, _CHUNK), jnp.float32),
    scratch_types=[
        pltpu.VMEM((_RPW,), jnp.int32),
        pltpu.VMEM((_RPW, _CHUNK), jnp.float32),
        pltpu.SemaphoreType.DMA,
    ],
)(_sc_gather_kernel)


C_BLK = 2048
N_COL_BLK = (CLASSES + C_BLK - 1) // C_BLK
N_HALF = 25
_B_FULL = 48 - N_HALF
_B_TAIL = 48 - N_HALF


def _loss_kernel(tgt_ref, chunk_ref, predA_ref, predB_ref, out_ref, acc_ref):
    i = pl.program_id(0)
    xA = predA_ref[...]
    xB = predB_ref[...]

    def partials(xe, xs):
        sumexp = jnp.sum(jnp.exp(xe), axis=1, keepdims=True)
        sumpred = jnp.sum(xs, axis=1, keepdims=True)
        return jnp.concatenate([sumexp, sumpred], axis=1)

    def accumulate(partial):
        @pl.when(i == 0)
        def _init():
            acc_ref[...] = partial

        @pl.when(i > 0)
        def _accum():
            acc_ref[...] += partial

    @pl.when(i < _B_TAIL)
    def _both_full():
        accumulate(partials(xA, xA) + partials(xB, xB))

    @pl.when(i == _B_TAIL)
    def _b_tail():
        cols = (jax.lax.broadcasted_iota(jnp.int32, xB.shape, 1)
                + (i + N_HALF) * C_BLK)
        valid = cols < CLASSES
        accumulate(partials(xA, xA)
                   + partials(jnp.where(valid, xB, -jnp.inf),
                              jnp.where(valid, xB, 0.0)))

    @pl.when(i == N_HALF - 1)
    def _a_only_and_finalize():
        accumulate(partials(xA, xA))
        acc = acc_ref[...]
        t = tgt_ref[...]
        lane = t - jnp.minimum(t & -_CHUNK, CLASSES - _CHUNK)
        lanes = jax.lax.broadcasted_iota(jnp.int32, (N_ROWS, _CHUNK), 1)
        tgtval = jnp.sum(jnp.where(lanes == lane, chunk_ref[...], 0.0),
                         axis=1, keepdims=True)
        rows = (jnp.log(acc[:, 0:1]) - EPS * acc[:, 1:2]
                - (CONFIDENCE - EPS) * tgtval)
        out_ref[0, 0] = jnp.sum(rows) / N_ROWS


@jax.jit
def _run(pred, target):
    tgt32 = target.astype(jnp.int32)
    chunks = _sc_gather(pred, tgt32)
    out = pl.pallas_call(
        _loss_kernel,
        grid=(N_HALF,),
        in_specs=[
            pl.BlockSpec((N_ROWS, 1), lambda i: (0, 0)),
            pl.BlockSpec((N_ROWS, _CHUNK), lambda i: (0, 0)),
            pl.BlockSpec((N_ROWS, C_BLK), lambda i: (0, i)),
            pl.BlockSpec((N_ROWS, C_BLK),
                         lambda i: (0, jnp.minimum(i + N_HALF,
                                                   N_COL_BLK - 1))),
        ],
        out_specs=pl.BlockSpec((1, 1), lambda i: (0, 0),
                               memory_space=pltpu.SMEM),
        out_shape=jax.ShapeDtypeStruct((1, 1), jnp.float32),
        scratch_shapes=[pltpu.VMEM((N_ROWS, 2), jnp.float32)],
        compiler_params=pltpu.CompilerParams(
            dimension_semantics=("arbitrary",),
        ),
    )(tgt32.reshape(N_ROWS, 1), chunks, pred, pred)
    return out[0, 0]


def kernel(pred, target):
    return _run(pred, target)
